# baseline (speedup 1.0000x reference)
import functools

import jax
import jax.numpy as jnp
from jax import lax
from jax.experimental import pallas as pl
from jax.experimental.pallas import tpu as pltpu
from jax.experimental.pallas import tpu_sc as plsc

H = 64
NC, NS = 2, 16
NW = NC * NS
LC = 4
UNROLL = 4


def _rsqrt(v):
    i = lax.bitcast_convert_type(v, jnp.int32)
    i = jnp.int32(0x5F3759DF) - lax.shift_right_logical(i, 1)
    y = lax.bitcast_convert_type(i, jnp.float32)
    for _ in range(3):
        y = y * (jnp.float32(1.5) - jnp.float32(0.5) * v * y * y)
    return y


def _sc_body(wid2, fid2, wtab, ftab, gamma_in, beta_in, out3,
             idx_w, idx_f, buf_w, buf_f, gam_v, bet_v, sem):
    w = lax.axis_index("s") * NC + lax.axis_index("c")
    n_rows = wid2.shape[0]
    rows_per_w = n_rows // NW
    n_chunks = rows_per_w // LC

    pltpu.sync_copy(gamma_in, gam_v)
    pltpu.sync_copy(beta_in, bet_v)
    gvec = [gam_v[pl.ds(16 * k, 16)] for k in range(4)]
    bvec = [bet_v[pl.ds(16 * k, 16)] for k in range(4)]

    lanes = lax.iota(jnp.int32, 16)
    perms = [lanes ^ st for st in (8, 4, 2, 1)]

    def allsum(v):
        for p in perms:
            v = v + v.at[p].get(mode="promise_in_bounds")
        return v

    def chunk_body(c, _):
        r0 = w * rows_per_w + c * LC
        pltpu.sync_copy(wid2.at[pl.ds(r0, LC)], idx_w)
        pltpu.sync_copy(fid2.at[pl.ds(r0, LC)], idx_f)
        descs = []
        for li in range(LC):
            descs.append(pltpu.async_copy(
                wtab.at[idx_w.at[li]], buf_w.at[li], sem))
            descs.append(pltpu.async_copy(
                ftab.at[idx_f.at[li]], buf_f.at[li], sem))
        for d in descs:
            d.wait()

        def tok_body(i, _):
            for uu in range(UNROLL):
                t = i * UNROLL + uu
                for li in range(LC):
                    x = [buf_w[li, t, pl.ds(16 * k, 16)]
                         + buf_f[li, t, pl.ds(16 * k, 16)] for k in range(4)]
                    s = allsum((x[0] + x[1]) + (x[2] + x[3]))
                    q = allsum((x[0] * x[0] + x[1] * x[1])
                               + (x[2] * x[2] + x[3] * x[3]))
                    u = s * jnp.float32(1.0 / H)
                    var = q * jnp.float32(1.0 / H) - u * u
                    r = _rsqrt(var + jnp.float32(1e-12))
                    for k in range(4):
                        buf_w[li, t, pl.ds(16 * k, 16)] = (
                            (x[k] - u) * r * gvec[k] + bvec[k])
            return 0

        lax.fori_loop(0, 128 // UNROLL, tok_body, 0)
        pltpu.sync_copy(buf_w, out3.at[pl.ds(r0, LC), :, pl.ds(0, H)])
        return 0

    lax.fori_loop(0, n_chunks, chunk_body, 0)


@functools.partial(jax.jit, static_argnums=(6,))
def _sc_embed(wid2, fid2, wtab, ftab, gamma, beta, n_rows):
    mesh = plsc.VectorSubcoreMesh(core_axis_name="c", subcore_axis_name="s")
    return pl.kernel(
        _sc_body,
        out_type=jax.ShapeDtypeStruct((n_rows, 128, 128), jnp.float32),
        mesh=mesh,
        scratch_types=[
            pltpu.VMEM((LC, 128), jnp.int32),
            pltpu.VMEM((LC, 128), jnp.int32),
            pltpu.VMEM((LC, 128, H), jnp.float32),
            pltpu.VMEM((LC, 128, H), jnp.float32),
            pltpu.VMEM((H,), jnp.float32),
            pltpu.VMEM((H,), jnp.float32),
            pltpu.SemaphoreType.DMA,
        ],
        compiler_params=pltpu.CompilerParams(use_tc_tiling_on_sc=False),
    )(wid2, fid2, wtab, ftab, gamma, beta)


def kernel(word_ids, age_ids, seg_ids, posi_ids, word_table, seg_table,
           age_table, posi_table, gamma, beta):
    B, L = word_ids.shape
    segv, h = seg_table.shape
    agev = age_table.shape[0]
    posv = posi_table.shape[0]
    n_rows = B * L // 128
    ftab = (seg_table[:, None, None, :] + age_table[None, :, None, :]
            + posi_table[None, None, :, :]).reshape(segv * agev * posv, h)
    wid2 = word_ids.astype(jnp.int32).reshape(n_rows, 128)
    fid2 = ((seg_ids.astype(jnp.int32) * agev + age_ids.astype(jnp.int32))
            * posv + posi_ids.astype(jnp.int32)).reshape(n_rows, 128)
    out3 = _sc_embed(wid2, fid2, word_table, ftab, gamma, beta, n_rows)
    return lax.optimization_barrier(out3[:, :, :h].reshape(B, L, h))


# speedup vs baseline: 1.2256x; 1.2256x over previous
import functools

import jax
import jax.numpy as jnp
from jax import lax
from jax.experimental import pallas as pl
from jax.experimental.pallas import tpu as pltpu
from jax.experimental.pallas import tpu_sc as plsc

H = 64
NC, NS = 2, 16
NW = NC * NS
LC = 4
UNROLL = 4


def _rsqrt(v):
    i = lax.bitcast_convert_type(v, jnp.int32)
    i = jnp.int32(0x5F3759DF) - lax.shift_right_logical(i, 1)
    y = lax.bitcast_convert_type(i, jnp.float32)
    for _ in range(3):
        y = y * (jnp.float32(1.5) - jnp.float32(0.5) * v * y * y)
    return y


def _sc_body(wid2, fid2, wtab, ftab, gamma_in, beta_in, out3,
             idx_w, idx_f, buf, gam_v, bet_v, gsem, asem, osem0, osem1):
    w = lax.axis_index("s") * NC + lax.axis_index("c")
    n_rows = wid2.shape[0]
    rows_per_w = n_rows // NW
    n_chunks = rows_per_w // LC
    osems = (osem0, osem1)

    pltpu.sync_copy(gamma_in, gam_v)
    pltpu.sync_copy(beta_in, bet_v)
    gvec = [gam_v[pl.ds(16 * k, 16)] for k in range(4)]
    bvec = [bet_v[pl.ds(16 * k, 16)] for k in range(4)]

    lanes = lax.iota(jnp.int32, 16)
    perms = [lanes ^ st for st in (8, 4, 2, 1)]

    def allsum(v):
        for p in perms:
            v = v + v.at[p].get(mode="promise_in_bounds")
        return v

    def fire_word(c, b):
        r0 = w * rows_per_w + c * LC
        pltpu.sync_copy(wid2.at[pl.ds(r0, LC)], idx_w.at[b])
        pltpu.sync_copy(fid2.at[pl.ds(r0, LC)], idx_f.at[b])
        for li in range(LC):
            pltpu.async_copy(wtab.at[idx_w.at[b, li]], buf.at[b, li], gsem)

    def fire_add(b):
        for li in range(LC):
            pltpu.async_copy(ftab.at[idx_f.at[b, li]], buf.at[b, li], asem,
                             add=True)

    def drain(b, sem):
        for li in range(LC):
            pltpu.make_async_copy(
                wtab.at[idx_w.at[b, li]], buf.at[b, li], sem).wait()

    def out_desc(c, b, sem):
        r0 = w * rows_per_w + c * LC
        return pltpu.make_async_copy(
            buf.at[b], out3.at[pl.ds(r0, LC), :, pl.ds(0, H)], sem)

    def compute(b):
        def tok_body(i, _):
            for uu in range(UNROLL):
                t = i * UNROLL + uu
                for li in range(LC):
                    x = [buf[b, li, t, pl.ds(16 * k, 16)] for k in range(4)]
                    s = allsum((x[0] + x[1]) + (x[2] + x[3]))
                    q = allsum((x[0] * x[0] + x[1] * x[1])
                               + (x[2] * x[2] + x[3] * x[3]))
                    u = s * jnp.float32(1.0 / H)
                    var = q * jnp.float32(1.0 / H) - u * u
                    r = _rsqrt(var + jnp.float32(1e-12))
                    for k in range(4):
                        buf[b, li, t, pl.ds(16 * k, 16)] = (
                            (x[k] - u) * r * gvec[k] + bvec[k])
            return 0

        lax.fori_loop(0, 128 // UNROLL, tok_body, 0)

    fire_word(0, 0)

    def pair_body(i, _):
        for b in range(2):
            c = i * 2 + b
            drain(b, gsem)
            fire_add(b)

            @pl.when(c + 1 < n_chunks)
            def _():
                @pl.when(c > 0)
                def _():
                    out_desc(c - 1, 1 - b, osems[1 - b]).wait()
                fire_word(c + 1, 1 - b)

            drain(b, asem)
            compute(b)
            out_desc(c, b, osems[b]).start()
        return 0

    lax.fori_loop(0, n_chunks // 2, pair_body, 0)
    out_desc(n_chunks - 2, 0, osems[0]).wait()
    out_desc(n_chunks - 1, 1, osems[1]).wait()


@functools.partial(jax.jit, static_argnums=(6,))
def _sc_embed(wid2, fid2, wtab, ftab, gamma, beta, n_rows):
    mesh = plsc.VectorSubcoreMesh(core_axis_name="c", subcore_axis_name="s")
    return pl.kernel(
        _sc_body,
        out_type=jax.ShapeDtypeStruct((n_rows, 128, 128), jnp.float32),
        mesh=mesh,
        scratch_types=[
            pltpu.VMEM((2, LC, 128), jnp.int32),
            pltpu.VMEM((2, LC, 128), jnp.int32),
            pltpu.VMEM((2, LC, 128, H), jnp.float32),
            pltpu.VMEM((H,), jnp.float32),
            pltpu.VMEM((H,), jnp.float32),
            pltpu.SemaphoreType.DMA,
            pltpu.SemaphoreType.DMA,
            pltpu.SemaphoreType.DMA,
            pltpu.SemaphoreType.DMA,
        ],
        compiler_params=pltpu.CompilerParams(use_tc_tiling_on_sc=False),
    )(wid2, fid2, wtab, ftab, gamma, beta)


def kernel(word_ids, age_ids, seg_ids, posi_ids, word_table, seg_table,
           age_table, posi_table, gamma, beta):
    B, L = word_ids.shape
    segv, h = seg_table.shape
    agev = age_table.shape[0]
    posv = posi_table.shape[0]
    n_rows = B * L // 128
    ftab = (seg_table[:, None, None, :] + age_table[None, :, None, :]
            + posi_table[None, None, :, :]).reshape(segv * agev * posv, h)
    wid2 = word_ids.astype(jnp.int32).reshape(n_rows, 128)
    fid2 = ((seg_ids.astype(jnp.int32) * agev + age_ids.astype(jnp.int32))
            * posv + posi_ids.astype(jnp.int32)).reshape(n_rows, 128)
    out3 = _sc_embed(wid2, fid2, word_table, ftab, gamma, beta, n_rows)
    return lax.optimization_barrier(out3[:, :, :h].reshape(B, L, h))


# speedup vs baseline: 1.2734x; 1.0390x over previous
import functools

import jax
import jax.numpy as jnp
from jax import lax
from jax.experimental import pallas as pl
from jax.experimental.pallas import tpu as pltpu
from jax.experimental.pallas import tpu_sc as plsc

H = 64
NC, NS = 2, 16
NW = NC * NS
LC = 4
UNROLL = 4


def _rsqrt(v):
    i = lax.bitcast_convert_type(v, jnp.int32)
    i = jnp.int32(0x5F3759DF) - lax.shift_right_logical(i, 1)
    y = lax.bitcast_convert_type(i, jnp.float32)
    for _ in range(3):
        y = y * (jnp.float32(1.5) - jnp.float32(0.5) * v * y * y)
    return y


def _sc_body(wid2, fid2, wtab, ftab, gamma_in, beta_in, out3,
             idx_w, idx_f, buf, gam_v, bet_v, gsem, asem, osem0, osem1, osem2):
    w = lax.axis_index("s") * NC + lax.axis_index("c")
    n_rows = wid2.shape[0]
    rows_per_w = n_rows // NW
    n_chunks = rows_per_w // LC
    osems = (osem0, osem1, osem2)

    pltpu.sync_copy(gamma_in, gam_v)
    pltpu.sync_copy(beta_in, bet_v)
    gvec = [gam_v[pl.ds(16 * k, 16)] for k in range(4)]
    bvec = [bet_v[pl.ds(16 * k, 16)] for k in range(4)]

    lanes = lax.iota(jnp.int32, 16)
    perms = [lanes ^ st for st in (8, 4, 2, 1)]

    def allsum(v):
        for p in perms:
            v = v + v.at[p].get(mode="promise_in_bounds")
        return v

    def fire_word(c, b):
        r0 = w * rows_per_w + c * LC
        pltpu.sync_copy(wid2.at[pl.ds(r0, LC)], idx_w.at[b])
        pltpu.sync_copy(fid2.at[pl.ds(r0, LC)], idx_f.at[b])
        for li in range(LC):
            pltpu.async_copy(wtab.at[idx_w.at[b, li]], buf.at[b, li], gsem)

    def fire_add(b):
        for li in range(LC):
            pltpu.async_copy(ftab.at[idx_f.at[b, li]], buf.at[b, li], asem,
                             add=True)

    def drain(b, sem):
        for li in range(LC):
            pltpu.make_async_copy(
                wtab.at[idx_w.at[b, li]], buf.at[b, li], sem).wait()

    def out_desc(c, b, sem):
        r0 = w * rows_per_w + c * LC
        return pltpu.make_async_copy(
            buf.at[b], out3.at[pl.ds(r0, LC), :, pl.ds(0, H)], sem)

    def compute(b):
        def tok_body(i, _):
            for uu in range(UNROLL):
                t = i * UNROLL + uu
                for li in range(LC):
                    x = [buf[b, li, t, pl.ds(16 * k, 16)] for k in range(4)]
                    s = allsum((x[0] + x[1]) + (x[2] + x[3]))
                    q = allsum((x[0] * x[0] + x[1] * x[1])
                               + (x[2] * x[2] + x[3] * x[3]))
                    u = s * jnp.float32(1.0 / H)
                    var = q * jnp.float32(1.0 / H) - u * u
                    r = _rsqrt(var + jnp.float32(1e-12))
                    for k in range(4):
                        buf[b, li, t, pl.ds(16 * k, 16)] = (
                            (x[k] - u) * r * gvec[k] + bvec[k])
            return 0

        lax.fori_loop(0, 128 // UNROLL, tok_body, 0)

    fire_word(0, 0)
    drain(0, gsem)
    fire_add(0)
    fire_word(1, 1)

    def tri_body(i, _):
        for j in range(3):
            c = i * 3 + j
            s0, s1, s2 = j % 3, (j + 1) % 3, (j + 2) % 3

            def reuse_and_fire():
                out_desc(c - 1, s2, osems[s2]).wait()
                fire_word(c + 2, s2)

            if j == 0:
                @pl.when(i > 0)
                def _():
                    reuse_and_fire()

                @pl.when(i == 0)
                def _():
                    fire_word(c + 2, s2)
            else:
                reuse_and_fire()

            drain(s1, gsem)
            fire_add(s1)
            drain(s0, asem)
            compute(s0)
            out_desc(c, s0, osems[s0]).start()
        return 0

    lax.fori_loop(0, (n_chunks - 2) // 3, tri_body, 0)
    c = n_chunks - 2
    out_desc(c - 1, 2, osems[2]).wait()
    drain(1, gsem)
    fire_add(1)
    drain(0, asem)
    compute(0)
    out_desc(c, 0, osems[0]).start()
    drain(1, asem)
    compute(1)
    out_desc(c + 1, 1, osems[1]).start()
    out_desc(c, 0, osems[0]).wait()
    out_desc(c + 1, 1, osems[1]).wait()


@functools.partial(jax.jit, static_argnums=(6,))
def _sc_embed(wid2, fid2, wtab, ftab, gamma, beta, n_rows):
    mesh = plsc.VectorSubcoreMesh(core_axis_name="c", subcore_axis_name="s")
    return pl.kernel(
        _sc_body,
        out_type=jax.ShapeDtypeStruct((n_rows, 128, 128), jnp.float32),
        mesh=mesh,
        scratch_types=[
            pltpu.VMEM((3, LC, 128), jnp.int32),
            pltpu.VMEM((3, LC, 128), jnp.int32),
            pltpu.VMEM((3, LC, 128, H), jnp.float32),
            pltpu.VMEM((H,), jnp.float32),
            pltpu.VMEM((H,), jnp.float32),
            pltpu.SemaphoreType.DMA,
            pltpu.SemaphoreType.DMA,
            pltpu.SemaphoreType.DMA,
            pltpu.SemaphoreType.DMA,
            pltpu.SemaphoreType.DMA,
        ],
        compiler_params=pltpu.CompilerParams(use_tc_tiling_on_sc=False),
    )(wid2, fid2, wtab, ftab, gamma, beta)


def kernel(word_ids, age_ids, seg_ids, posi_ids, word_table, seg_table,
           age_table, posi_table, gamma, beta):
    B, L = word_ids.shape
    segv, h = seg_table.shape
    agev = age_table.shape[0]
    posv = posi_table.shape[0]
    n_rows = B * L // 128
    ftab = (seg_table[:, None, None, :] + age_table[None, :, None, :]
            + posi_table[None, None, :, :]).reshape(segv * agev * posv, h)
    wid2 = word_ids.astype(jnp.int32).reshape(n_rows, 128)
    fid2 = ((seg_ids.astype(jnp.int32) * agev + age_ids.astype(jnp.int32))
            * posv + posi_ids.astype(jnp.int32)).reshape(n_rows, 128)
    out3 = _sc_embed(wid2, fid2, word_table, ftab, gamma, beta, n_rows)
    return lax.optimization_barrier(out3[:, :, :h].reshape(B, L, h))
